# baseline (speedup 1.0000x reference)
import functools

import jax
import jax.numpy as jnp
from jax import lax
from jax.experimental import pallas as pl
from jax.experimental.pallas import tpu as pltpu
from jax.experimental.pallas import tpu_sc as plsc

_NUM_CLASSES = 100000
_DIM = 128
_BATCH = 16384
_NC, _NS, _L = 2, 16, 16
_NW = _NC * _NS
_ROWS_W = _BATCH // _NW
_CHUNK = 128
_NCHUNK = _ROWS_W // _CHUNK
_YROWS = _BATCH // _CHUNK
_HIST_W = 6272
_HIST_PAD = _NS * _HIST_W


def _sc_body(y2d, hidden, centers, d_out, c_out,
             hist, y1_v, ones_v, y2_v, cnt_v, cen_v, hid_v, d_v, zbuf, sem):
  cid = lax.axis_index("c")
  sid = lax.axis_index("s")
  wid = sid * _NC + cid

  def _zero(i, carry):
    zbuf[pl.ds(i * _L, _L)] = jnp.zeros((_L,), jnp.float32)
    return carry
  lax.fori_loop(0, _HIST_W // _L, _zero, 0)
  pltpu.sync_copy(zbuf, hist.at[pl.ds(sid * _HIST_W, _HIST_W)])

  for j in range(8):
    ones_v[pl.ds(j * _L, _L)] = jnp.ones((_L,), jnp.float32)
  pltpu.sync_copy(y2d.at[pl.ds(sid * 8, 8)], y1_v)
  plsc.subcore_barrier()

  for j in range(8):
    pltpu.sync_copy(ones_v, hist.at[y1_v.at[j]], add=True)
  plsc.subcore_barrier()

  base_row = wid * _NCHUNK
  pltpu.sync_copy(y2d.at[pl.ds(base_row, _NCHUNK)], y2_v)
  for j in range(_NCHUNK):
    pltpu.async_copy(hist.at[y2_v.at[j]], cnt_v.at[j], sem).wait()

  for j in range(_NCHUNK):
    r0 = wid * _ROWS_W + j * _CHUNK
    pltpu.async_copy(centers.at[y2_v.at[j]], cen_v, sem).wait()
    pltpu.sync_copy(hidden.at[pl.ds(r0, _CHUNK)], hid_v)

    def _row(r, carry):
      acc = jnp.zeros((_L,), jnp.float32)
      for q in range(_DIM // _L):
        h = hid_v[r, pl.ds(q * _L, _L)]
        c = cen_v[r, pl.ds(q * _L, _L)]
        dif = h - c
        acc = acc + dif * dif
      d_v[r] = acc
      return carry
    lax.fori_loop(0, _CHUNK, _row, 0)
    pltpu.sync_copy(d_v, d_out.at[pl.ds(r0, _CHUNK)])

  pltpu.sync_copy(cnt_v, c_out.at[pl.ds(base_row, _NCHUNK)])


_sc_kernel = functools.partial(
    pl.kernel,
    out_type=(
        jax.ShapeDtypeStruct((_BATCH, _L), jnp.float32),
        jax.ShapeDtypeStruct((_YROWS, _CHUNK), jnp.float32),
    ),
    mesh=plsc.VectorSubcoreMesh(core_axis_name="c", subcore_axis_name="s"),
    scratch_types=[
        pltpu.VMEM_SHARED((_HIST_PAD,), jnp.float32),
        pltpu.VMEM((8, _CHUNK), jnp.int32),
        pltpu.VMEM((_CHUNK,), jnp.float32),
        pltpu.VMEM((_NCHUNK, _CHUNK), jnp.int32),
        pltpu.VMEM((_NCHUNK, _CHUNK), jnp.float32),
        pltpu.VMEM((_CHUNK, _DIM), jnp.float32),
        pltpu.VMEM((_CHUNK, _DIM), jnp.float32),
        pltpu.VMEM((_CHUNK, _L), jnp.float32),
        pltpu.VMEM((_HIST_W,), jnp.float32),
        pltpu.SemaphoreType.DMA,
    ],
)(_sc_body)


def _finish_body(d_ref, c_ref, o_ref):
  d = jnp.sum(d_ref[...], axis=2)
  s = jnp.sum(d / (c_ref[...] + 1.0))
  o_ref[0, 0] = 0.5 * s / (s + 1e-6)


def kernel(y, hidden, centers):
  y2d = jnp.reshape(y, (_YROWS, _CHUNK))
  d, c = _sc_kernel(y2d, hidden, centers)
  d3 = jnp.reshape(d, (_YROWS, _CHUNK, _L))
  out = pl.pallas_call(
      _finish_body,
      out_shape=jax.ShapeDtypeStruct((1, 1), jnp.float32),
      out_specs=pl.BlockSpec(memory_space=pltpu.SMEM),
  )(d3, c)
  return out[0, 0]
